# baseline (speedup 1.0000x reference)
import jax
import jax.numpy as jnp
from jax.experimental import pallas as pl
from jax.experimental.pallas import tpu as pltpu


def _sublane_mean_f32(xt, inv_s):
    S = xt.shape[0]
    ch = next((c for c in (512, 256, 128, 64, 32, 16, 8) if S % c == 0), None)
    if ch is None:
        pooled = jnp.sum(xt.astype(jnp.float32), axis=0, keepdims=True)
        return pooled * inv_s
    accs = []
    for k in range(S // ch):
        c = xt[k * ch:(k + 1) * ch, :]
        if c.dtype != jnp.float32:
            c = c.astype(jnp.float32)
        if len(accs) < 8:
            accs.append(c)
        else:
            accs[k % 8] = accs[k % 8] + c
    acc = accs[0]
    for a in accs[1:]:
        acc = acc + a
    return jnp.sum(acc, axis=0, keepdims=True) * inv_s


def kernel(x, w1, b1, w2, b2):
    B, C, D, H, W = x.shape
    hid = w1.shape[0]
    S = D * H * W
    inv_s = 1.0 / float(S)

    x_bsc = jnp.transpose(x, (0, 2, 3, 4, 1)).reshape(B, S, C)
    wpack = jnp.concatenate([
        w1,
        jnp.transpose(w2),
        jnp.pad(b1.reshape(1, hid), ((0, 0), (0, C - hid))),
        b2.reshape(1, C),
    ], axis=0)

    contract_last = (((1,), (1,)), ((), ()))
    Bt = 4 if B % 4 == 0 else (2 if B % 2 == 0 else 1)

    def se_kernel(x_ref, wp_ref, o_ref):
        xt = x_ref[...]
        rows = [_sublane_mean_f32(xt[i], inv_s) for i in range(Bt)]
        pooled = rows[0] if Bt == 1 else jnp.concatenate(rows, axis=0)
        w1m = wp_ref[0:hid]
        w2t = wp_ref[hid:2 * hid]
        b1r = wp_ref[2 * hid:2 * hid + 1, 0:hid]
        b2r = wp_ref[2 * hid + 1:2 * hid + 2]
        h = jax.lax.dot_general(pooled, w1m, contract_last,
                                preferred_element_type=jnp.float32)
        h = h + b1r
        h = jnp.where(h >= 0.0, h, 0.01 * h)
        y = jax.lax.dot_general(h, w2t, (((1,), (0,)), ((), ())),
                                preferred_element_type=jnp.float32)
        y = y + b2r
        g = jax.nn.sigmoid(y)
        if g.dtype != xt.dtype:
            g = g.astype(xt.dtype)
        o_ref[...] = (xt * g[:, None, :]).astype(o_ref.dtype)

    out_bsc = pl.pallas_call(
        se_kernel,
        out_shape=jax.ShapeDtypeStruct((B, S, C), x.dtype),
        grid=(B // Bt,),
        in_specs=[
            pl.BlockSpec((Bt, S, C), lambda b: (b, 0, 0)),
            pl.BlockSpec((2 * hid + 2, C), lambda b: (0, 0)),
        ],
        out_specs=pl.BlockSpec((Bt, S, C), lambda b: (b, 0, 0)),
        compiler_params=pltpu.CompilerParams(
            dimension_semantics=("parallel",),
            allow_input_fusion=[False, True],
            vmem_limit_bytes=64 * 1024 * 1024),
    )(x_bsc, wpack)

    return jnp.transpose(out_bsc.reshape(B, D, H, W, C), (0, 4, 1, 2, 3))


# speedup vs baseline: 1.0050x; 1.0050x over previous
import jax
import jax.numpy as jnp
from jax.experimental import pallas as pl
from jax.experimental.pallas import tpu as pltpu


def _sublane_mean_f32(xt, inv_s):
    S = xt.shape[0]
    ch = next((c for c in (512, 256, 128, 64, 32, 16, 8) if S % c == 0), None)
    if ch is None:
        pooled = jnp.sum(xt.astype(jnp.float32), axis=0, keepdims=True)
        return pooled * inv_s
    accs = []
    for k in range(S // ch):
        c = xt[k * ch:(k + 1) * ch, :]
        if c.dtype != jnp.float32:
            c = c.astype(jnp.float32)
        if len(accs) < 8:
            accs.append(c)
        else:
            accs[k % 8] = accs[k % 8] + c
    acc = accs[0]
    for a in accs[1:]:
        acc = acc + a
    return jnp.sum(acc, axis=0, keepdims=True) * inv_s


def kernel(x, w1, b1, w2, b2):
    B, C, D, H, W = x.shape
    hid = w1.shape[0]
    S = D * H * W
    inv_s = 1.0 / float(S)

    x_bsc = jnp.transpose(x, (0, 2, 3, 4, 1)).reshape(B, S, C)
    wpack = jnp.concatenate([
        w1,
        jnp.transpose(w2),
        jnp.pad(b1.reshape(1, hid), ((0, 0), (0, C - hid))),
        b2.reshape(1, C),
    ], axis=0)

    contract_last = (((1,), (1,)), ((), ()))
    Bt = 4 if B % 4 == 0 else (2 if B % 2 == 0 else 1)

    def se_kernel(x_ref, wp_ref, o_ref):
        xt = x_ref[...]
        rows = [_sublane_mean_f32(xt[i], inv_s) for i in range(Bt)]
        pooled = rows[0] if Bt == 1 else jnp.concatenate(rows, axis=0)
        w1m = wp_ref[0:hid]
        w2t = wp_ref[hid:2 * hid]
        b1r = wp_ref[2 * hid:2 * hid + 1, 0:hid]
        b2r = wp_ref[2 * hid + 1:2 * hid + 2]
        h = jax.lax.dot_general(pooled, w1m, contract_last,
                                preferred_element_type=jnp.float32)
        h = h + b1r
        h = jnp.where(h >= 0.0, h, 0.01 * h)
        y = jax.lax.dot_general(h, w2t, (((1,), (0,)), ((), ())),
                                preferred_element_type=jnp.float32)
        y = y + b2r
        g = jax.nn.sigmoid(y)
        if g.dtype != xt.dtype:
            g = g.astype(xt.dtype)
        o_ref[...] = (xt * g[:, None, :]).astype(o_ref.dtype)

    out_bsc = pl.pallas_call(
        se_kernel,
        out_shape=jax.ShapeDtypeStruct((B, S, C), x.dtype),
        grid=(B // Bt,),
        in_specs=[
            pl.BlockSpec((Bt, S, C), lambda b: (b, 0, 0)),
            pl.BlockSpec((2 * hid + 2, C), lambda b: (0, 0)),
        ],
        out_specs=pl.BlockSpec((Bt, S, C), lambda b: (b, 0, 0)),
        compiler_params=pltpu.CompilerParams(
            dimension_semantics=("parallel",),
            vmem_limit_bytes=64 * 1024 * 1024),
    )(x_bsc, wpack)

    return jnp.transpose(out_bsc.reshape(B, D, H, W, C), (0, 4, 1, 2, 3))
